# baseline (speedup 1.0000x reference)
import functools

import jax
import jax.numpy as jnp
from jax.experimental import pallas as pl
from jax.experimental.pallas import tpu as pltpu

T = 2048
D = 768
E = 8
K = 2
FF = 768

TILE = 256
NT = T // TILE


def _moe_body(x_ref, gamma_ref, beta_ref, wr_ref, w1_ref, b1_ref, w2_ref,
              b2_ref, y_ref, h_s, g_s):
    e = pl.program_id(0)
    t = pl.program_id(1)
    ts = pl.ds(t * TILE, TILE)

    @pl.when(e == 0)
    def _():
        xb = x_ref[...]
        mu = jnp.mean(xb, axis=-1, keepdims=True)
        var = jnp.mean((xb - mu) ** 2, axis=-1, keepdims=True)
        h = (xb - mu) * jax.lax.rsqrt(var + 1e-6) * gamma_ref[...][None, :] \
            + beta_ref[...][None, :]
        h_s[ts, :] = h
        logits = jnp.dot(h, wr_ref[...], preferred_element_type=jnp.float32)
        p = jax.nn.softmax(logits, axis=-1)
        m1 = jnp.max(p, axis=-1, keepdims=True)
        p2 = jnp.where(p >= m1, -jnp.inf, p)
        m2 = jnp.max(p2, axis=-1, keepdims=True)
        g = jnp.where(p >= m2, p, 0.0)
        g = g / (jnp.sum(g, axis=-1, keepdims=True) + 1e-9)
        g_s[ts, :] = g

    h = h_s[ts, :].astype(jnp.bfloat16)
    w1 = w1_ref[0].astype(jnp.bfloat16)
    hid = jax.nn.gelu(
        jnp.dot(h, w1, preferred_element_type=jnp.float32)
        + b1_ref[0]).astype(jnp.bfloat16)
    w2 = w2_ref[0].astype(jnp.bfloat16)
    eo = jnp.dot(hid, w2, preferred_element_type=jnp.float32) \
        + b2_ref[0]
    g_tile = g_s[ts, :]
    lane = jax.lax.broadcasted_iota(jnp.int32, (1, E), 1)
    g_e = jnp.sum(jnp.where(lane == e, g_tile, 0.0), axis=1, keepdims=True)
    contrib = g_e * eo

    @pl.when(e == 0)
    def _():
        y_ref[ts, :] = x_ref[...] + contrib

    @pl.when(e != 0)
    def _():
        y_ref[ts, :] += contrib


@functools.partial(jax.jit, static_argnames=())
def kernel(x, gamma, beta, W_router, W1, b1, W2, b2):
    y = pl.pallas_call(
        _moe_body,
        grid=(E, NT),
        in_specs=[
            pl.BlockSpec((TILE, D), lambda e, t: (t, 0)),
            pl.BlockSpec((D,), lambda e, t: (0,)),
            pl.BlockSpec((D,), lambda e, t: (0,)),
            pl.BlockSpec((D, E), lambda e, t: (0, 0)),
            pl.BlockSpec((1, D, FF), lambda e, t: (e, 0, 0)),
            pl.BlockSpec((1, 1, FF), lambda e, t: (e, 0, 0)),
            pl.BlockSpec((1, FF, D), lambda e, t: (e, 0, 0)),
            pl.BlockSpec((1, 1, D), lambda e, t: (e, 0, 0)),
        ],
        out_specs=pl.BlockSpec((T, D), lambda e, t: (0, 0)),
        out_shape=jax.ShapeDtypeStruct((T, D), jnp.float32),
        scratch_shapes=[
            pltpu.VMEM((T, D), jnp.float32),
            pltpu.VMEM((T, E), jnp.float32),
        ],
    )(x, gamma, beta, W_router, W1, b1.reshape(E, 1, FF), W2,
      b2.reshape(E, 1, D))
    return (y, jnp.float32(0.0))


# speedup vs baseline: 1.0613x; 1.0613x over previous
import functools

import jax
import jax.numpy as jnp
from jax.experimental import pallas as pl
from jax.experimental.pallas import tpu as pltpu

T = 2048
D = 768
E = 8
K = 2
FF = 768

TILE = 256
NT = T // TILE


def _moe_body(x_ref, gamma_ref, beta_ref, wr_ref, w1_ref, b1_ref, w2_ref,
              b2_ref, y_ref, h_s, g_s, w1b_s, w2b_s):
    e = pl.program_id(0)
    t = pl.program_id(1)
    ts = pl.ds(t * TILE, TILE)

    @pl.when(e == 0)
    def _():
        xb = x_ref[...]
        mu = jnp.mean(xb, axis=-1, keepdims=True)
        var = jnp.mean((xb - mu) ** 2, axis=-1, keepdims=True)
        h = (xb - mu) * jax.lax.rsqrt(var + 1e-6) * gamma_ref[...][None, :] \
            + beta_ref[...][None, :]
        h_s[ts, :] = h
        logits = jnp.dot(h, wr_ref[...], preferred_element_type=jnp.float32)
        p = jax.nn.softmax(logits, axis=-1)
        m1 = jnp.max(p, axis=-1, keepdims=True)
        p2 = jnp.where(p >= m1, -jnp.inf, p)
        m2 = jnp.max(p2, axis=-1, keepdims=True)
        g = jnp.where(p >= m2, p, 0.0)
        g = g / (jnp.sum(g, axis=-1, keepdims=True) + 1e-9)
        g_s[ts, :] = g

    @pl.when(t == 0)
    def _():
        w1b_s[...] = w1_ref[0].astype(jnp.bfloat16)
        w2b_s[...] = w2_ref[0].astype(jnp.bfloat16)

    h = h_s[ts, :].astype(jnp.bfloat16)
    hid = jax.nn.gelu(
        jnp.dot(h, w1b_s[...], preferred_element_type=jnp.float32)
        + b1_ref[0]).astype(jnp.bfloat16)
    eo = jnp.dot(hid, w2b_s[...], preferred_element_type=jnp.float32) \
        + b2_ref[0]
    g_tile = g_s[ts, :]
    lane = jax.lax.broadcasted_iota(jnp.int32, (1, E), 1)
    g_e = jnp.sum(jnp.where(lane == e, g_tile, 0.0), axis=1, keepdims=True)
    contrib = g_e * eo

    @pl.when(e == 0)
    def _():
        y_ref[ts, :] = x_ref[...] + contrib

    @pl.when(e != 0)
    def _():
        y_ref[ts, :] += contrib


@functools.partial(jax.jit, static_argnames=())
def kernel(x, gamma, beta, W_router, W1, b1, W2, b2):
    y = pl.pallas_call(
        _moe_body,
        grid=(E, NT),
        in_specs=[
            pl.BlockSpec((TILE, D), lambda e, t: (t, 0)),
            pl.BlockSpec((D,), lambda e, t: (0,)),
            pl.BlockSpec((D,), lambda e, t: (0,)),
            pl.BlockSpec((D, E), lambda e, t: (0, 0)),
            pl.BlockSpec((1, D, FF), lambda e, t: (e, 0, 0)),
            pl.BlockSpec((1, 1, FF), lambda e, t: (e, 0, 0)),
            pl.BlockSpec((1, FF, D), lambda e, t: (e, 0, 0)),
            pl.BlockSpec((1, 1, D), lambda e, t: (e, 0, 0)),
        ],
        out_specs=pl.BlockSpec((T, D), lambda e, t: (0, 0)),
        out_shape=jax.ShapeDtypeStruct((T, D), jnp.float32),
        scratch_shapes=[
            pltpu.VMEM((T, D), jnp.float32),
            pltpu.VMEM((T, E), jnp.float32),
            pltpu.VMEM((D, FF), jnp.bfloat16),
            pltpu.VMEM((FF, D), jnp.bfloat16),
        ],
    )(x, gamma, beta, W_router, W1, b1.reshape(E, 1, FF), W2,
      b2.reshape(E, 1, D))
    return (y, jnp.float32(0.0))
